# baseline (speedup 1.0000x reference)
import functools

import jax
import jax.numpy as jnp
from jax import lax
from jax.experimental import pallas as pl
from jax.experimental.pallas import tpu as pltpu
from jax.experimental.pallas import tpu_sc as plsc

_N = 10000
_E = 320000
_D = 128
_C = 128
_G = 3
_NSUB = 16
_NCHUNKS = _E // _C
_GPT = _NCHUNKS // _G // _NSUB
_NTAIL = _NCHUNKS - _NSUB * _GPT * _G
_RPT = (_N // _NSUB) & ~7
_TAIL = _N - _NSUB * _RPT
_GC = _G * _C


def _sc_body(v_x, v_ui, v_di, v_ua, v_da,
             e_x, e_ui, e_di, e_ua, e_da,
             t_x, t_ui, t_di, t_ua, t_da,
             zeros_hbm,
             o_v, o_e, o_t,
             acc, idx_s, idx_d, big,
             sem_is, sem_id, sem_g, sem_a, sem_sc):
    cid = lax.axis_index("c")
    sid = lax.axis_index("s")
    row0 = sid * _RPT

    def run_dir(x_hbm, idx_hbm, attr_hbm):
        def body(p, carry):
            e0 = (sid + _NSUB * p) * _GC
            ci_s = pltpu.async_copy(idx_hbm.at[0, pl.ds(e0, _GC)],
                                    idx_s, sem_is)
            ci_d = pltpu.async_copy(idx_hbm.at[1, pl.ds(e0, _GC)],
                                    idx_d, sem_id)
            ci_s.wait()
            for b in range(_G):
                pltpu.async_copy(x_hbm.at[idx_s.at[pl.ds(b * _C, _C)]],
                                 big.at[pl.ds(b * _C, _C)], sem_g)
            pltpu.make_async_copy(attr_hbm.at[pl.ds(0, _GC)], big,
                                  sem_g).wait()
            ci_d.wait()
            for b in range(_G):
                pltpu.async_copy(big.at[pl.ds(b * _C, _C)],
                                 acc.at[idx_d.at[pl.ds(b * _C, _C)]],
                                 sem_sc, add=True)
            pltpu.make_async_copy(attr_hbm.at[pl.ds(0, _GC)], big,
                                  sem_sc).wait()
            pltpu.async_copy(attr_hbm.at[pl.ds(e0, _GC)], big, sem_a).wait()
            for b in range(_G):
                pltpu.async_copy(big.at[pl.ds(b * _C, _C)],
                                 acc.at[idx_d.at[pl.ds(b * _C, _C)]],
                                 sem_sc, add=True)
            pltpu.make_async_copy(attr_hbm.at[pl.ds(0, _GC)], big,
                                  sem_sc).wait()
            return carry

        lax.fori_loop(0, _GPT, body, 0)

        @pl.when(sid < _NTAIL)
        def _():
            e0 = (_NSUB * _GPT * _G + sid) * _C
            pltpu.sync_copy(idx_hbm.at[0, pl.ds(e0, _C)],
                            idx_s.at[pl.ds(0, _C)])
            pltpu.sync_copy(idx_hbm.at[1, pl.ds(e0, _C)],
                            idx_d.at[pl.ds(0, _C)])
            pltpu.async_copy(x_hbm.at[idx_s.at[pl.ds(0, _C)]],
                             big.at[pl.ds(0, _C)], sem_g).wait()
            pltpu.sync_copy(big.at[pl.ds(0, _C)],
                            acc.at[idx_d.at[pl.ds(0, _C)]], add=True)
            pltpu.sync_copy(attr_hbm.at[pl.ds(e0, _C)], big.at[pl.ds(0, _C)])
            pltpu.sync_copy(big.at[pl.ds(0, _C)],
                            acc.at[idx_d.at[pl.ds(0, _C)]], add=True)

    def level(x_hbm, ui, di, ua, da, out_hbm):
        pltpu.sync_copy(zeros_hbm.at[pl.ds(row0, _RPT)],
                        acc.at[pl.ds(row0, _RPT)])

        @pl.when(sid == _NSUB - 1)
        def _():
            pltpu.sync_copy(zeros_hbm.at[pl.ds(_NSUB * _RPT, _TAIL)],
                            acc.at[pl.ds(_NSUB * _RPT, _TAIL)])

        plsc.subcore_barrier()

        @pl.when(cid == 0)
        def _():
            run_dir(x_hbm, ui, ua)

        @pl.when(cid == 1)
        def _():
            run_dir(x_hbm, di, da)

        plsc.subcore_barrier()
        pltpu.sync_copy(acc.at[pl.ds(row0, _RPT)],
                        out_hbm.at[cid, pl.ds(row0, _RPT)])

        @pl.when(sid == _NSUB - 1)
        def _():
            pltpu.sync_copy(acc.at[pl.ds(_NSUB * _RPT, _TAIL)],
                            out_hbm.at[cid, pl.ds(_NSUB * _RPT, _TAIL)])

        plsc.subcore_barrier()

    level(v_x, v_ui, v_di, v_ua, v_da, o_v)
    level(e_x, e_ui, e_di, e_ua, e_da, o_e)
    level(t_x, t_ui, t_di, t_ua, t_da, o_t)


_sc_mp = functools.partial(
    pl.kernel,
    out_type=[jax.ShapeDtypeStruct((2, _N, _D), jnp.float32)] * 3,
    mesh=plsc.VectorSubcoreMesh(core_axis_name="c", subcore_axis_name="s"),
    scratch_types=[
        pltpu.VMEM_SHARED((_N, _D), jnp.float32),
        pltpu.VMEM((_GC,), jnp.int32),
        pltpu.VMEM((_GC,), jnp.int32),
        pltpu.VMEM((_GC, _D), jnp.float32),
    ] + [pltpu.SemaphoreType.DMA] * 5,
)(_sc_body)


def _combine_body(pv, pe, pt, ov, oe, ot):
    ov[...] = pv[0] + pv[1]
    oe[...] = pe[0] + pe[1]
    ot[...] = pt[0] + pt[1]


def _combine(pv, pe, pt):
    b = 1000
    return pl.pallas_call(
        _combine_body,
        grid=(_N // b,),
        in_specs=[pl.BlockSpec((2, b, _D), lambda i: (0, i, 0))] * 3,
        out_specs=[pl.BlockSpec((b, _D), lambda i: (i, 0))] * 3,
        out_shape=[jax.ShapeDtypeStruct((_N, _D), jnp.float32)] * 3,
    )(pv, pe, pt)


def kernel(v_x, v_up_index, v_down_index, v_up_attr, v_down_attr,
           e_x, e_up_index, e_down_index, e_up_attr, e_down_attr,
           t_x, t_up_index, t_down_index, t_up_attr, t_down_attr):
    zeros = jnp.zeros((_N, _D), jnp.float32)
    pv, pe, pt = _sc_mp(
        v_x, v_up_index, v_down_index, v_up_attr, v_down_attr,
        e_x, e_up_index, e_down_index, e_up_attr, e_down_attr,
        t_x, t_up_index, t_down_index, t_up_attr, t_down_attr,
        zeros)
    return _combine(pv, pe, pt)


# speedup vs baseline: 1.1181x; 1.1181x over previous
import functools

import jax
import jax.numpy as jnp
from jax import lax
from jax.experimental import pallas as pl
from jax.experimental.pallas import tpu as pltpu
from jax.experimental.pallas import tpu_sc as plsc

_N = 10000
_E = 320000
_D = 128
_C = 128
_G = 2
_NSUB = 16
_NCHUNKS = _E // _C
_GPT = _NCHUNKS // _G // _NSUB
_NTAIL = _NCHUNKS - _NSUB * _GPT * _G
_RPT = (_N // _NSUB) & ~7
_TAIL = _N - _NSUB * _RPT
_GC = _G * _C


def _sc_body(v_x, v_ui, v_di, v_ua, v_da,
             e_x, e_ui, e_di, e_ua, e_da,
             t_x, t_ui, t_di, t_ua, t_da,
             zeros_hbm,
             o_v, o_e, o_t,
             acc, idx_s, idx_d, big, attrb,
             sem_is, sem_id, sem_g, sem_a, sem_s1, sem_s2):
    cid = lax.axis_index("c")
    sid = lax.axis_index("s")
    row0 = sid * _RPT

    def run_dir(x_hbm, idx_hbm, attr_hbm):
        def body(p, carry):
            e0 = (sid + _NSUB * p) * _GC
            a0 = pltpu.async_copy(attr_hbm.at[pl.ds(e0, _C)], attrb, sem_a)
            ci_s = pltpu.async_copy(idx_hbm.at[0, pl.ds(e0, _GC)],
                                    idx_s, sem_is)
            ci_d = pltpu.async_copy(idx_hbm.at[1, pl.ds(e0, _GC)],
                                    idx_d, sem_id)
            ci_s.wait()
            gs = [pltpu.async_copy(x_hbm.at[idx_s.at[pl.ds(b * _C, _C)]],
                                   big.at[pl.ds(b * _C, _C)], sem_g)
                  for b in range(_G)]
            ci_d.wait()
            gs[0].wait()
            pltpu.async_copy(big.at[pl.ds(0, _C)],
                             acc.at[idx_d.at[pl.ds(0, _C)]],
                             sem_s1, add=True)
            a0.wait()
            s20 = pltpu.async_copy(attrb, acc.at[idx_d.at[pl.ds(0, _C)]],
                                   sem_s2, add=True)
            gs[1].wait()
            pltpu.async_copy(big.at[pl.ds(_C, _C)],
                             acc.at[idx_d.at[pl.ds(_C, _C)]],
                             sem_s1, add=True)
            s20.wait()
            a1 = pltpu.async_copy(attr_hbm.at[pl.ds(e0 + _C, _C)],
                                  attrb, sem_a)
            a1.wait()
            s21 = pltpu.async_copy(attrb, acc.at[idx_d.at[pl.ds(_C, _C)]],
                                   sem_s2, add=True)
            pltpu.make_async_copy(attr_hbm.at[pl.ds(0, _GC)], big,
                                  sem_s1).wait()
            s21.wait()
            return carry

        lax.fori_loop(0, _GPT, body, 0)

        @pl.when(sid < _NTAIL)
        def _():
            e0 = (_NSUB * _GPT * _G + sid) * _C
            pltpu.sync_copy(idx_hbm.at[0, pl.ds(e0, _C)],
                            idx_s.at[pl.ds(0, _C)])
            pltpu.sync_copy(idx_hbm.at[1, pl.ds(e0, _C)],
                            idx_d.at[pl.ds(0, _C)])
            pltpu.async_copy(x_hbm.at[idx_s.at[pl.ds(0, _C)]],
                             big.at[pl.ds(0, _C)], sem_g).wait()
            pltpu.sync_copy(big.at[pl.ds(0, _C)],
                            acc.at[idx_d.at[pl.ds(0, _C)]], add=True)
            pltpu.sync_copy(attr_hbm.at[pl.ds(e0, _C)], attrb)
            pltpu.sync_copy(attrb,
                            acc.at[idx_d.at[pl.ds(0, _C)]], add=True)

    def level(x_hbm, ui, di, ua, da, out_hbm):
        pltpu.sync_copy(zeros_hbm.at[pl.ds(row0, _RPT)],
                        acc.at[pl.ds(row0, _RPT)])

        @pl.when(sid == _NSUB - 1)
        def _():
            pltpu.sync_copy(zeros_hbm.at[pl.ds(_NSUB * _RPT, _TAIL)],
                            acc.at[pl.ds(_NSUB * _RPT, _TAIL)])

        plsc.subcore_barrier()

        @pl.when(cid == 0)
        def _():
            run_dir(x_hbm, ui, ua)

        @pl.when(cid == 1)
        def _():
            run_dir(x_hbm, di, da)

        plsc.subcore_barrier()
        pltpu.sync_copy(acc.at[pl.ds(row0, _RPT)],
                        out_hbm.at[cid, pl.ds(row0, _RPT)])

        @pl.when(sid == _NSUB - 1)
        def _():
            pltpu.sync_copy(acc.at[pl.ds(_NSUB * _RPT, _TAIL)],
                            out_hbm.at[cid, pl.ds(_NSUB * _RPT, _TAIL)])

        plsc.subcore_barrier()

    level(v_x, v_ui, v_di, v_ua, v_da, o_v)
    level(e_x, e_ui, e_di, e_ua, e_da, o_e)
    level(t_x, t_ui, t_di, t_ua, t_da, o_t)


_sc_mp = functools.partial(
    pl.kernel,
    out_type=[jax.ShapeDtypeStruct((2, _N, _D), jnp.float32)] * 3,
    mesh=plsc.VectorSubcoreMesh(core_axis_name="c", subcore_axis_name="s"),
    scratch_types=[
        pltpu.VMEM_SHARED((_N, _D), jnp.float32),
        pltpu.VMEM((_GC,), jnp.int32),
        pltpu.VMEM((_GC,), jnp.int32),
        pltpu.VMEM((_GC, _D), jnp.float32),
        pltpu.VMEM((_C, _D), jnp.float32),
    ] + [pltpu.SemaphoreType.DMA] * 6,
)(_sc_body)


def _combine_body(pv, pe, pt, ov, oe, ot):
    ov[...] = pv[0] + pv[1]
    oe[...] = pe[0] + pe[1]
    ot[...] = pt[0] + pt[1]


def _combine(pv, pe, pt):
    b = 1000
    return pl.pallas_call(
        _combine_body,
        grid=(_N // b,),
        in_specs=[pl.BlockSpec((2, b, _D), lambda i: (0, i, 0))] * 3,
        out_specs=[pl.BlockSpec((b, _D), lambda i: (i, 0))] * 3,
        out_shape=[jax.ShapeDtypeStruct((_N, _D), jnp.float32)] * 3,
    )(pv, pe, pt)


def kernel(v_x, v_up_index, v_down_index, v_up_attr, v_down_attr,
           e_x, e_up_index, e_down_index, e_up_attr, e_down_attr,
           t_x, t_up_index, t_down_index, t_up_attr, t_down_attr):
    zeros = jnp.zeros((_N, _D), jnp.float32)
    pv, pe, pt = _sc_mp(
        v_x, v_up_index, v_down_index, v_up_attr, v_down_attr,
        e_x, e_up_index, e_down_index, e_up_attr, e_down_attr,
        t_x, t_up_index, t_down_index, t_up_attr, t_down_attr,
        zeros)
    return _combine(pv, pe, pt)


# speedup vs baseline: 1.1249x; 1.0061x over previous
import functools

import jax
import jax.numpy as jnp
from jax import lax
from jax.experimental import pallas as pl
from jax.experimental.pallas import tpu as pltpu
from jax.experimental.pallas import tpu_sc as plsc

_N = 10000
_E = 320000
_D = 128
_C = 128
_G = 2
_NSUB = 16
_NCHUNKS = _E // _C
_GPT = _NCHUNKS // _G // _NSUB
_NTAIL = _NCHUNKS - _NSUB * _GPT * _G
_RPT = (_N // _NSUB) & ~7
_TAIL = _N - _NSUB * _RPT
_GC = _G * _C


def _sc_body(v_x, v_ui, v_di, v_ua, v_da,
             e_x, e_ui, e_di, e_ua, e_da,
             t_x, t_ui, t_di, t_ua, t_da,
             zeros_hbm,
             o_v, o_e, o_t,
             acc, idxb, big, attrb,
             sem_is, sem_g, sem_a, sem_s1, sem_s2):
    cid = lax.axis_index("c")
    sid = lax.axis_index("s")
    row0 = sid * _RPT

    def run_dir(x_hbm, idx_hbm, attr_hbm):
        def body(p, carry):
            e0 = (sid + _NSUB * p) * _GC
            a0 = pltpu.async_copy(attr_hbm.at[pl.ds(e0, _C)], attrb, sem_a)
            ci = pltpu.async_copy(idx_hbm.at[:, pl.ds(e0, _GC)],
                                  idxb, sem_is)
            ci.wait()
            gs = [pltpu.async_copy(x_hbm.at[idxb.at[0, pl.ds(b * _C, _C)]],
                                   big.at[pl.ds(b * _C, _C)], sem_g)
                  for b in range(_G)]
            gs[0].wait()
            pltpu.async_copy(big.at[pl.ds(0, _C)],
                             acc.at[idxb.at[1, pl.ds(0, _C)]],
                             sem_s1, add=True)
            a0.wait()
            s20 = pltpu.async_copy(attrb, acc.at[idxb.at[1, pl.ds(0, _C)]],
                                   sem_s2, add=True)
            gs[1].wait()
            pltpu.async_copy(big.at[pl.ds(_C, _C)],
                             acc.at[idxb.at[1, pl.ds(_C, _C)]],
                             sem_s1, add=True)
            s20.wait()
            a1 = pltpu.async_copy(attr_hbm.at[pl.ds(e0 + _C, _C)],
                                  attrb, sem_a)
            a1.wait()
            s21 = pltpu.async_copy(attrb, acc.at[idxb.at[1, pl.ds(_C, _C)]],
                                   sem_s2, add=True)
            pltpu.make_async_copy(attr_hbm.at[pl.ds(0, _GC)], big,
                                  sem_s1).wait()
            s21.wait()
            return carry

        lax.fori_loop(0, _GPT, body, 0)

        @pl.when(sid < _NTAIL)
        def _():
            e0 = (_NSUB * _GPT * _G + sid) * _C
            pltpu.sync_copy(idx_hbm.at[:, pl.ds(e0, _C)],
                            idxb.at[:, pl.ds(0, _C)])
            pltpu.async_copy(x_hbm.at[idxb.at[0, pl.ds(0, _C)]],
                             big.at[pl.ds(0, _C)], sem_g).wait()
            pltpu.sync_copy(big.at[pl.ds(0, _C)],
                            acc.at[idxb.at[1, pl.ds(0, _C)]], add=True)
            pltpu.sync_copy(attr_hbm.at[pl.ds(e0, _C)], attrb)
            pltpu.sync_copy(attrb,
                            acc.at[idxb.at[1, pl.ds(0, _C)]], add=True)

    def level(x_hbm, ui, di, ua, da, out_hbm):
        pltpu.sync_copy(zeros_hbm.at[pl.ds(row0, _RPT)],
                        acc.at[pl.ds(row0, _RPT)])

        @pl.when(sid == _NSUB - 1)
        def _():
            pltpu.sync_copy(zeros_hbm.at[pl.ds(_NSUB * _RPT, _TAIL)],
                            acc.at[pl.ds(_NSUB * _RPT, _TAIL)])

        plsc.subcore_barrier()

        @pl.when(cid == 0)
        def _():
            run_dir(x_hbm, ui, ua)

        @pl.when(cid == 1)
        def _():
            run_dir(x_hbm, di, da)

        plsc.subcore_barrier()
        pltpu.sync_copy(acc.at[pl.ds(row0, _RPT)],
                        out_hbm.at[cid, pl.ds(row0, _RPT)])

        @pl.when(sid == _NSUB - 1)
        def _():
            pltpu.sync_copy(acc.at[pl.ds(_NSUB * _RPT, _TAIL)],
                            out_hbm.at[cid, pl.ds(_NSUB * _RPT, _TAIL)])

        plsc.subcore_barrier()

    level(v_x, v_ui, v_di, v_ua, v_da, o_v)
    level(e_x, e_ui, e_di, e_ua, e_da, o_e)
    level(t_x, t_ui, t_di, t_ua, t_da, o_t)


_sc_mp = functools.partial(
    pl.kernel,
    out_type=[jax.ShapeDtypeStruct((2, _N, _D), jnp.float32)] * 3,
    mesh=plsc.VectorSubcoreMesh(core_axis_name="c", subcore_axis_name="s"),
    scratch_types=[
        pltpu.VMEM_SHARED((_N, _D), jnp.float32),
        pltpu.VMEM((2, _GC), jnp.int32),
        pltpu.VMEM((_GC, _D), jnp.float32),
        pltpu.VMEM((_C, _D), jnp.float32),
    ] + [pltpu.SemaphoreType.DMA] * 5,
)(_sc_body)


def _combine_body(pv, pe, pt, ov, oe, ot):
    ov[...] = pv[0] + pv[1]
    oe[...] = pe[0] + pe[1]
    ot[...] = pt[0] + pt[1]


def _combine(pv, pe, pt):
    b = 1000
    return pl.pallas_call(
        _combine_body,
        grid=(_N // b,),
        in_specs=[pl.BlockSpec((2, b, _D), lambda i: (0, i, 0))] * 3,
        out_specs=[pl.BlockSpec((b, _D), lambda i: (i, 0))] * 3,
        out_shape=[jax.ShapeDtypeStruct((_N, _D), jnp.float32)] * 3,
    )(pv, pe, pt)


def kernel(v_x, v_up_index, v_down_index, v_up_attr, v_down_attr,
           e_x, e_up_index, e_down_index, e_up_attr, e_down_attr,
           t_x, t_up_index, t_down_index, t_up_attr, t_down_attr):
    zeros = jnp.zeros((_N, _D), jnp.float32)
    pv, pe, pt = _sc_mp(
        v_x, v_up_index, v_down_index, v_up_attr, v_down_attr,
        e_x, e_up_index, e_down_index, e_up_attr, e_down_attr,
        t_x, t_up_index, t_down_index, t_up_attr, t_down_attr,
        zeros)
    return _combine(pv, pe, pt)


# speedup vs baseline: 1.1271x; 1.0019x over previous
import functools

import jax
import jax.numpy as jnp
from jax import lax
from jax.experimental import pallas as pl
from jax.experimental.pallas import tpu as pltpu
from jax.experimental.pallas import tpu_sc as plsc

_N = 10000
_E = 320000
_D = 128
_C = 128
_G = 2
_NSUB = 16
_NCHUNKS = _E // _C
_GPT = _NCHUNKS // _G // _NSUB
_NTAIL = _NCHUNKS - _NSUB * _GPT * _G
_RPT = (_N // _NSUB) & ~7
_TAIL = _N - _NSUB * _RPT
_GC = _G * _C


def _sc_body(v_x, v_ui, v_di, v_ua, v_da,
             e_x, e_ui, e_di, e_ua, e_da,
             t_x, t_ui, t_di, t_ua, t_da,
             zeros_hbm,
             o_v, o_e, o_t,
             acc, idxb, big, attrb,
             sem_is, sem_g, sem_a, sem_s1, sem_s2):
    cid = lax.axis_index("c")
    sid = lax.axis_index("s")
    row0 = sid * _RPT

    def run_dir(x_hbm, idx_hbm, attr_hbm):
        def body(p, carry):
            e0 = (sid + _NSUB * p) * _GC
            a0 = pltpu.async_copy(attr_hbm.at[pl.ds(e0, _C)], attrb, sem_a)
            ci = pltpu.async_copy(idx_hbm.at[:, pl.ds(e0, _GC)],
                                  idxb, sem_is)
            ci.wait()
            gs = [pltpu.async_copy(x_hbm.at[idxb.at[0, pl.ds(b * _C, _C)]],
                                   big.at[pl.ds(b * _C, _C)], sem_g)
                  for b in range(_G)]
            gs[0].wait()
            s10 = pltpu.async_copy(big.at[pl.ds(0, _C)],
                                   acc.at[idxb.at[1, pl.ds(0, _C)]],
                                   sem_s1, add=True)
            a0.wait()
            s20 = pltpu.async_copy(attrb, acc.at[idxb.at[1, pl.ds(0, _C)]],
                                   sem_s2, add=True)
            gs[1].wait()
            s11 = pltpu.async_copy(big.at[pl.ds(_C, _C)],
                                   acc.at[idxb.at[1, pl.ds(_C, _C)]],
                                   sem_s1, add=True)
            s10.wait()
            a1 = pltpu.async_copy(attr_hbm.at[pl.ds(e0 + _C, _C)],
                                  big.at[pl.ds(0, _C)], sem_a)
            a1.wait()
            s21 = pltpu.async_copy(big.at[pl.ds(0, _C)],
                                   acc.at[idxb.at[1, pl.ds(_C, _C)]],
                                   sem_s2, add=True)
            s11.wait()
            s20.wait()
            s21.wait()
            return carry

        lax.fori_loop(0, _GPT, body, 0)

        @pl.when(sid < _NTAIL)
        def _():
            e0 = (_NSUB * _GPT * _G + sid) * _C
            pltpu.sync_copy(idx_hbm.at[:, pl.ds(e0, _C)],
                            idxb.at[:, pl.ds(0, _C)])
            pltpu.async_copy(x_hbm.at[idxb.at[0, pl.ds(0, _C)]],
                             big.at[pl.ds(0, _C)], sem_g).wait()
            pltpu.sync_copy(big.at[pl.ds(0, _C)],
                            acc.at[idxb.at[1, pl.ds(0, _C)]], add=True)
            pltpu.sync_copy(attr_hbm.at[pl.ds(e0, _C)], attrb)
            pltpu.sync_copy(attrb,
                            acc.at[idxb.at[1, pl.ds(0, _C)]], add=True)

    def level(x_hbm, ui, di, ua, da, out_hbm):
        pltpu.sync_copy(zeros_hbm.at[pl.ds(row0, _RPT)],
                        acc.at[pl.ds(row0, _RPT)])

        @pl.when(sid == _NSUB - 1)
        def _():
            pltpu.sync_copy(zeros_hbm.at[pl.ds(_NSUB * _RPT, _TAIL)],
                            acc.at[pl.ds(_NSUB * _RPT, _TAIL)])

        plsc.subcore_barrier()

        @pl.when(cid == 0)
        def _():
            run_dir(x_hbm, ui, ua)

        @pl.when(cid == 1)
        def _():
            run_dir(x_hbm, di, da)

        plsc.subcore_barrier()
        pltpu.sync_copy(acc.at[pl.ds(row0, _RPT)],
                        out_hbm.at[cid, pl.ds(row0, _RPT)])

        @pl.when(sid == _NSUB - 1)
        def _():
            pltpu.sync_copy(acc.at[pl.ds(_NSUB * _RPT, _TAIL)],
                            out_hbm.at[cid, pl.ds(_NSUB * _RPT, _TAIL)])

        plsc.subcore_barrier()

    level(v_x, v_ui, v_di, v_ua, v_da, o_v)
    level(e_x, e_ui, e_di, e_ua, e_da, o_e)
    level(t_x, t_ui, t_di, t_ua, t_da, o_t)


_sc_mp = functools.partial(
    pl.kernel,
    out_type=[jax.ShapeDtypeStruct((2, _N, _D), jnp.float32)] * 3,
    mesh=plsc.VectorSubcoreMesh(core_axis_name="c", subcore_axis_name="s"),
    scratch_types=[
        pltpu.VMEM_SHARED((_N, _D), jnp.float32),
        pltpu.VMEM((2, _GC), jnp.int32),
        pltpu.VMEM((_GC, _D), jnp.float32),
        pltpu.VMEM((_C, _D), jnp.float32),
    ] + [pltpu.SemaphoreType.DMA] * 5,
)(_sc_body)


def _combine_body(pv, pe, pt, ov, oe, ot):
    ov[...] = pv[0] + pv[1]
    oe[...] = pe[0] + pe[1]
    ot[...] = pt[0] + pt[1]


def _combine(pv, pe, pt):
    b = 1000
    return pl.pallas_call(
        _combine_body,
        grid=(_N // b,),
        in_specs=[pl.BlockSpec((2, b, _D), lambda i: (0, i, 0))] * 3,
        out_specs=[pl.BlockSpec((b, _D), lambda i: (i, 0))] * 3,
        out_shape=[jax.ShapeDtypeStruct((_N, _D), jnp.float32)] * 3,
    )(pv, pe, pt)


def kernel(v_x, v_up_index, v_down_index, v_up_attr, v_down_attr,
           e_x, e_up_index, e_down_index, e_up_attr, e_down_attr,
           t_x, t_up_index, t_down_index, t_up_attr, t_down_attr):
    zeros = jnp.zeros((_N, _D), jnp.float32)
    pv, pe, pt = _sc_mp(
        v_x, v_up_index, v_down_index, v_up_attr, v_down_attr,
        e_x, e_up_index, e_down_index, e_up_attr, e_down_attr,
        t_x, t_up_index, t_down_index, t_up_attr, t_down_attr,
        zeros)
    return _combine(pv, pe, pt)
